# baseline (speedup 1.0000x reference)
import functools

import jax
import jax.numpy as jnp
from jax import lax
from jax.experimental import pallas as pl
from jax.experimental.pallas import tpu as pltpu
from jax.experimental.pallas import tpu_sc as plsc

LANES = 16
DIM = 64
BLK = 128
WIN = 4
NFULL = 7812
IDX_CHUNK = 12800
SEL_CAP = 2112
SEL_ALLOC = 2128
WCAP = 96
WPAD = 112


def _splat(x, dtype=jnp.int32):
    return jnp.full((LANES,), x, dtype)


@functools.lru_cache(maxsize=None)
def _build_gather(n_idx: int, vocab: int):
    info = plsc.get_sparse_core_info()
    nw = info.num_cores * info.num_subcores
    n_win = NFULL // (WIN * nw) + 2
    mesh = plsc.VectorSubcoreMesh(core_axis_name="c", subcore_axis_name="s")

    @functools.partial(
        pl.kernel,
        mesh=mesh,
        out_type=jax.ShapeDtypeStruct((n_idx, 2 * DIM), jnp.float32),
        compiler_params=pltpu.CompilerParams(
            use_tc_tiling_on_sc=False, needs_layout_passes=False),
        scratch_types=[
            pltpu.VMEM((IDX_CHUNK,), jnp.int32),
            pltpu.VMEM((SEL_ALLOC,), jnp.int32),
            pltpu.VMEM((SEL_ALLOC,), jnp.int32),
            pltpu.VMEM((DIM, WIN * BLK), jnp.float32),
            pltpu.VMEM((DIM, DIM), jnp.float32),
            pltpu.VMEM((WPAD,), jnp.int32),
            pltpu.VMEM((WPAD,), jnp.int32),
            pltpu.VMEM((WCAP, 2 * DIM), jnp.float32),
            pltpu.SemaphoreType.DMA,
        ],
    )
    def gather(tableT, idx_hbm, out_hbm, icbuf, sel_i, sel_n, staged,
               staged_t, wrel, wn, rowbuf, osem):
        wid = lax.axis_index("s") * info.num_cores + lax.axis_index("c")
        lo = (NFULL * wid) // nw
        hi = (NFULL * (wid + 1)) // nw
        is_last = wid == nw - 1
        hi_sel = jnp.where(is_last, NFULL + 1, hi)
        iota = lax.iota(jnp.int32, LANES)

        def scan_group(g, carry):
            pos, nbase = carry
            v = icbuf[pl.ds(g * LANES, LANES)]
            blk = lax.shift_right_logical(v, 7)
            m = (blk >= _splat(lo)) & (blk < _splat(hi_sel))
            pc = plsc.cumsum(m.astype(jnp.int32))
            dst = jnp.where(m, _splat(pos) + pc - 1, _splat(SEL_ALLOC - 1))
            plsc.store_scatter(sel_i, [dst], v)
            nvec = _splat(nbase) + _splat(g * LANES) + iota
            plsc.store_scatter(sel_n, [dst], nvec)
            return pos + jnp.max(pc), nbase

        pos = jnp.int32(0)
        for c in range(n_idx // IDX_CHUNK):
            pltpu.sync_copy(idx_hbm.at[pl.ds(c * IDX_CHUNK, IDX_CHUNK)], icbuf)
            pos, _ = lax.fori_loop(
                0, IDX_CHUNK // LANES, scan_group,
                (pos, jnp.int32(c * IDX_CHUNK)))

        def compact_and_extract(start, hiw, c0, src_ref, src_w):
            def initw(q, carry):
                wn[pl.ds(q * LANES, LANES)] = _splat(-1)
                return carry
            lax.fori_loop(0, WPAD // LANES, initw, 0)

            def cgrp(g, wpos):
                lm = (_splat(g * LANES) + iota) < _splat(pos)
                v = sel_i[pl.ds(g * LANES, LANES)]
                nv = sel_n[pl.ds(g * LANES, LANES)]
                blk = lax.shift_right_logical(v, 7)
                m = lm & (blk >= _splat(start)) & (blk < _splat(hiw))
                pc = plsc.cumsum(m.astype(jnp.int32))
                dst = jnp.where(m, _splat(wpos) + pc - 1, _splat(WPAD - 1))
                plsc.store_scatter(wrel, [dst], v - _splat(c0))
                plsc.store_scatter(wn, [dst], nv)
                return wpos + jnp.max(pc)

            wpos = lax.fori_loop(0, SEL_CAP // LANES, cgrp, jnp.int32(0))

            for q in range(WCAP // LANES):
                @pl.when(q * LANES < wpos)
                def _():
                    jv = _splat(q * LANES) + iota
                    rel = plsc.load_gather(wrel, [jv]) & _splat(src_w - 1)
                    for c in range(DIM):
                        vals = plsc.load_gather(src_ref, [_splat(c), rel])
                        plsc.store_scatter(rowbuf, [jv, _splat(c)], vals)

            pltpu.async_copy(
                rowbuf,
                out_hbm.at[plsc.Indices(wn.at[pl.ds(0, WCAP)],
                                        ignored_value=-1)],
                osem).wait()

        def window(i, carry):
            start = lo + i * WIN
            @pl.when(start < hi)
            def _():
                c0 = pl.multiple_of(
                    jnp.minimum(start * BLK, (NFULL - WIN) * BLK), BLK)
                pltpu.sync_copy(tableT.at[:, pl.ds(c0, WIN * BLK)], staged)
                hiw = jnp.minimum(start + WIN, hi)
                compact_and_extract(start, hiw, c0, staged, WIN * BLK)
            return carry

        lax.fori_loop(0, n_win, window, 0)

        @pl.when(is_last)
        def _():
            pltpu.sync_copy(tableT.at[:, pl.ds(NFULL * BLK, DIM)], staged_t)
            compact_and_extract(jnp.int32(NFULL), jnp.int32(NFULL + 1),
                                jnp.int32(NFULL * BLK), staged_t, DIM)

    return gather


def kernel(kwids, kw_dist_adj, mask, word_embed_table):
    vocab, dim = word_embed_table.shape
    idx = kwids.reshape(-1)
    gather = _build_gather(idx.shape[0], vocab)
    out128 = gather(word_embed_table.T, idx)
    kw_embed = out128[:, :dim].reshape(kwids.shape + (dim,))
    return (kw_embed, kw_dist_adj, mask)


# speedup vs baseline: 1.0510x; 1.0510x over previous
import functools

import jax
import jax.numpy as jnp
from jax import lax
from jax.experimental import pallas as pl
from jax.experimental.pallas import tpu as pltpu
from jax.experimental.pallas import tpu_sc as plsc

LANES = 16
DIM = 64
BLK = 128
WIN = 4
NFULL = 7812
IDX_CHUNK = 12800
SEL_CAP = 2112
SEL_ALLOC = 2128
WCAP = 96
WPAD = 112


def _splat(x, dtype=jnp.int32):
    return jnp.full((LANES,), x, dtype)


@functools.lru_cache(maxsize=None)
def _build_gather(n_idx: int, vocab: int):
    info = plsc.get_sparse_core_info()
    nw = info.num_cores * info.num_subcores
    n_win = NFULL // (WIN * nw) + 2
    mesh = plsc.VectorSubcoreMesh(core_axis_name="c", subcore_axis_name="s")

    @functools.partial(
        pl.kernel,
        mesh=mesh,
        out_type=jax.ShapeDtypeStruct((n_idx, 2 * DIM), jnp.float32),
        compiler_params=pltpu.CompilerParams(
            use_tc_tiling_on_sc=False, needs_layout_passes=False),
        scratch_types=[
            pltpu.VMEM((IDX_CHUNK,), jnp.int32),
            pltpu.VMEM((SEL_ALLOC,), jnp.int32),
            pltpu.VMEM((SEL_ALLOC,), jnp.int32),
            pltpu.VMEM((DIM, WIN * BLK), jnp.float32),
            pltpu.VMEM((DIM, DIM), jnp.float32),
            pltpu.VMEM((WPAD,), jnp.int32),
            pltpu.VMEM((WPAD,), jnp.int32),
            pltpu.VMEM((WCAP, 2 * DIM), jnp.float32),
            pltpu.SemaphoreType.DMA,
        ],
    )
    def gather(tableT, idx_hbm, out_hbm, icbuf, sel_i, sel_n, staged,
               staged_t, wrel, wn, rowbuf, osem):
        wid = lax.axis_index("s") * info.num_cores + lax.axis_index("c")
        lo = (NFULL * wid) // nw
        hi = (NFULL * (wid + 1)) // nw
        is_last = wid == nw - 1
        hi_sel = jnp.where(is_last, NFULL + 1, hi)
        iota = lax.iota(jnp.int32, LANES)

        def scan_group(g, carry):
            pos, nbase = carry
            v = icbuf[pl.ds(g * LANES, LANES)]
            blk = lax.shift_right_logical(v, 7)
            m = (blk >= _splat(lo)) & (blk < _splat(hi_sel))
            pc = plsc.cumsum(m.astype(jnp.int32))
            dst = jnp.where(m, _splat(pos) + pc - 1, _splat(SEL_ALLOC - 1))
            plsc.store_scatter(sel_i, [dst], v)
            nvec = _splat(nbase) + _splat(g * LANES) + iota
            plsc.store_scatter(sel_n, [dst], nvec)
            return pos + jnp.max(pc), nbase

        pos = jnp.int32(0)
        for c in range(n_idx // IDX_CHUNK):
            pltpu.sync_copy(idx_hbm.at[pl.ds(c * IDX_CHUNK, IDX_CHUNK)], icbuf)
            pos, _ = lax.fori_loop(
                0, IDX_CHUNK // LANES, scan_group,
                (pos, jnp.int32(c * IDX_CHUNK)))

        def compact_and_extract(start, hiw, c0, src_ref, src_w):
            def initw(q, carry):
                wn[pl.ds(q * LANES, LANES)] = _splat(-1)
                return carry
            lax.fori_loop(0, WPAD // LANES, initw, 0)

            def cgrp(g, wpos):
                lm = (_splat(g * LANES) + iota) < _splat(pos)
                v = sel_i[pl.ds(g * LANES, LANES)]
                nv = sel_n[pl.ds(g * LANES, LANES)]
                blk = lax.shift_right_logical(v, 7)
                m = lm & (blk >= _splat(start)) & (blk < _splat(hiw))
                pc = plsc.cumsum(m.astype(jnp.int32))
                dst = jnp.where(m, _splat(wpos) + pc - 1, _splat(WPAD - 1))
                plsc.store_scatter(wrel, [dst], v - _splat(c0))
                plsc.store_scatter(wn, [dst], nv)
                return wpos + jnp.max(pc)

            wpos = lax.fori_loop(0, SEL_CAP // LANES, cgrp, jnp.int32(0))

            for q in range(WCAP // LANES):
                @pl.when(q * LANES < wpos)
                def _():
                    jv = _splat(q * LANES) + iota
                    rel = plsc.load_gather(wrel, [jv]) & _splat(src_w - 1)
                    for c in range(DIM):
                        vals = plsc.load_gather(src_ref, [_splat(c), rel])
                        plsc.store_scatter(rowbuf, [jv, _splat(c)], vals)

            pltpu.async_copy(
                rowbuf,
                out_hbm.at[plsc.Indices(wn.at[pl.ds(0, WCAP)],
                                        ignored_value=-1)],
                osem).wait()

        def window(i, carry):
            start = lo + i * WIN
            @pl.when(start < hi)
            def _():
                c0 = pl.multiple_of(
                    jnp.minimum(start * BLK, (NFULL - WIN) * BLK), BLK)
                pltpu.sync_copy(tableT.at[:, pl.ds(c0, WIN * BLK)], staged)
            return carry

        lax.fori_loop(0, n_win, window, 0)

        @pl.when(is_last)
        def _():
            pltpu.sync_copy(tableT.at[:, pl.ds(NFULL * BLK, DIM)], staged_t)
            compact_and_extract(jnp.int32(NFULL), jnp.int32(NFULL + 1),
                                jnp.int32(NFULL * BLK), staged_t, DIM)

    return gather


def kernel(kwids, kw_dist_adj, mask, word_embed_table):
    vocab, dim = word_embed_table.shape
    idx = kwids.reshape(-1)
    gather = _build_gather(idx.shape[0], vocab)
    out128 = gather(word_embed_table.T, idx)
    kw_embed = out128[:, :dim].reshape(kwids.shape + (dim,))
    return (kw_embed, kw_dist_adj, mask)


# speedup vs baseline: 1.0513x; 1.0003x over previous
import functools

import jax
import jax.numpy as jnp
from jax import lax
from jax.experimental import pallas as pl
from jax.experimental.pallas import tpu as pltpu
from jax.experimental.pallas import tpu_sc as plsc

LANES = 16
DIM = 64
BLK = 128
WIN = 4
NFULL = 7812
IDX_CHUNK = 12800
SEL_CAP = 2112
SEL_ALLOC = 2128
WCAP = 96
WPAD = 112


def _splat(x, dtype=jnp.int32):
    return jnp.full((LANES,), x, dtype)


@functools.lru_cache(maxsize=None)
def _build_gather(n_idx: int, vocab: int):
    info = plsc.get_sparse_core_info()
    nw = info.num_cores * info.num_subcores
    n_win = NFULL // (WIN * nw) + 2
    mesh = plsc.VectorSubcoreMesh(core_axis_name="c", subcore_axis_name="s")

    @functools.partial(
        pl.kernel,
        mesh=mesh,
        out_type=jax.ShapeDtypeStruct((n_idx, 2 * DIM), jnp.float32),
        compiler_params=pltpu.CompilerParams(
            use_tc_tiling_on_sc=False, needs_layout_passes=False),
        scratch_types=[
            pltpu.VMEM((IDX_CHUNK,), jnp.int32),
            pltpu.VMEM((SEL_ALLOC,), jnp.int32),
            pltpu.VMEM((SEL_ALLOC,), jnp.int32),
            pltpu.VMEM((DIM, WIN * BLK), jnp.float32),
            pltpu.VMEM((DIM, DIM), jnp.float32),
            pltpu.VMEM((WPAD,), jnp.int32),
            pltpu.VMEM((WPAD,), jnp.int32),
            pltpu.VMEM((WCAP, 2 * DIM), jnp.float32),
            pltpu.SemaphoreType.DMA,
            pltpu.SemaphoreType.DMA,
        ],
    )
    def gather(tableT, idx_hbm, out_hbm, icbuf, sel_i, sel_n, staged,
               staged_t, wrel, wn, rowbuf, osem, gsem):
        wid = lax.axis_index("s") * info.num_cores + lax.axis_index("c")
        lo = (NFULL * wid) // nw
        hi = (NFULL * (wid + 1)) // nw
        is_last = wid == nw - 1
        hi_sel = jnp.where(is_last, NFULL + 1, hi)
        iota = lax.iota(jnp.int32, LANES)

        def scan_group(g, carry):
            pos, nbase = carry
            v = icbuf[pl.ds(g * LANES, LANES)]
            blk = lax.shift_right_logical(v, 7)
            m = (blk >= _splat(lo)) & (blk < _splat(hi_sel))
            pc = plsc.cumsum(m.astype(jnp.int32))
            dst = jnp.where(m, _splat(pos) + pc - 1, _splat(SEL_ALLOC - 1))
            plsc.store_scatter(sel_i, [dst], v)
            nvec = _splat(nbase) + _splat(g * LANES) + iota
            plsc.store_scatter(sel_n, [dst], nvec)
            return pos + jnp.max(pc), nbase

        pos = jnp.int32(0)
        for c in range(n_idx // IDX_CHUNK):
            pltpu.sync_copy(idx_hbm.at[pl.ds(c * IDX_CHUNK, IDX_CHUNK)], icbuf)
            pos, _ = lax.fori_loop(
                0, IDX_CHUNK // LANES, scan_group,
                (pos, jnp.int32(c * IDX_CHUNK)))

        def compact_and_extract(start, hiw, c0, src_ref, src_w):
            def initw(q, carry):
                wn[pl.ds(q * LANES, LANES)] = _splat(-1)
                return carry
            lax.fori_loop(0, WPAD // LANES, initw, 0)

            def cgrp(g, wpos):
                lm = (_splat(g * LANES) + iota) < _splat(pos)
                v = sel_i[pl.ds(g * LANES, LANES)]
                nv = sel_n[pl.ds(g * LANES, LANES)]
                blk = lax.shift_right_logical(v, 7)
                m = lm & (blk >= _splat(start)) & (blk < _splat(hiw))
                pc = plsc.cumsum(m.astype(jnp.int32))
                dst = jnp.where(m, _splat(wpos) + pc - 1, _splat(WPAD - 1))
                plsc.store_scatter(wrel, [dst], v - _splat(c0))
                plsc.store_scatter(wn, [dst], nv)
                return wpos + jnp.max(pc)

            wpos = lax.fori_loop(0, SEL_CAP // LANES, cgrp, jnp.int32(0))

            for q in range(WCAP // LANES):
                @pl.when(q * LANES < wpos)
                def _():
                    jv = _splat(q * LANES) + iota
                    rel = plsc.load_gather(wrel, [jv]) & _splat(src_w - 1)
                    for c in range(DIM):
                        vals = plsc.load_gather(src_ref, [_splat(c), rel])
                        plsc.store_scatter(rowbuf, [jv, _splat(c)], vals)

            pltpu.async_copy(
                rowbuf,
                out_hbm.at[plsc.Indices(wn.at[pl.ds(0, WCAP)],
                                        ignored_value=-1)],
                osem).wait()

        def window(i, carry):
            start = lo + i * WIN
            @pl.when(start < hi)
            def _():
                c0 = pl.multiple_of(
                    jnp.minimum(start * BLK, (NFULL - WIN) * BLK), BLK)
                hs = [pltpu.async_copy(
                    tableT.at[c].at[pl.ds(c0, WIN * BLK)],
                    staged.at[c], gsem) for c in range(DIM)]
                for h in hs:
                    h.wait()
            return carry

        lax.fori_loop(0, n_win, window, 0)

        @pl.when(is_last)
        def _():
            pltpu.sync_copy(tableT.at[:, pl.ds(NFULL * BLK, DIM)], staged_t)
            compact_and_extract(jnp.int32(NFULL), jnp.int32(NFULL + 1),
                                jnp.int32(NFULL * BLK), staged_t, DIM)

    return gather


def kernel(kwids, kw_dist_adj, mask, word_embed_table):
    vocab, dim = word_embed_table.shape
    idx = kwids.reshape(-1)
    gather = _build_gather(idx.shape[0], vocab)
    out128 = gather(word_embed_table.T, idx)
    kw_embed = out128[:, :dim].reshape(kwids.shape + (dim,))
    return (kw_embed, kw_dist_adj, mask)


# speedup vs baseline: 3.6394x; 3.4617x over previous
import functools

import jax
import jax.numpy as jnp
from jax import lax
from jax.experimental import pallas as pl
from jax.experimental.pallas import tpu as pltpu
from jax.experimental.pallas import tpu_sc as plsc


@functools.lru_cache(maxsize=None)
def _build_gather(n_idx: int, dim: int, vocab: int):
    info = plsc.get_sparse_core_info()
    nw = info.num_cores * info.num_subcores
    assert n_idx % nw == 0
    per_w = n_idx // nw
    half = dim // 2
    mesh = plsc.VectorSubcoreMesh(core_axis_name="c", subcore_axis_name="s")

    @functools.partial(
        pl.kernel,
        mesh=mesh,
        out_type=(
            jax.ShapeDtypeStruct((n_idx, half), jnp.float32),
            jax.ShapeDtypeStruct((n_idx, half), jnp.float32),
        ),
        compiler_params=pltpu.CompilerParams(use_tc_tiling_on_sc=False),
        scratch_types=[
            pltpu.VMEM((per_w,), jnp.int32),
            pltpu.VMEM((per_w, half), jnp.float32),
            pltpu.VMEM((per_w, half), jnp.float32),
            pltpu.SemaphoreType.DMA,
            pltpu.SemaphoreType.DMA,
            pltpu.SemaphoreType.DMA,
        ],
    )
    def gather(t0, t1, idx_hbm, out0, out1, idx_v, rows0, rows1,
               s0, s1, so):
        wid = lax.axis_index("s") * info.num_cores + lax.axis_index("c")
        base = wid * per_w
        pltpu.sync_copy(idx_hbm.at[pl.ds(base, per_w)], idx_v)
        h0 = pltpu.async_copy(t0.at[idx_v], rows0, s0)
        h1 = pltpu.async_copy(t1.at[idx_v], rows1, s1)
        h0.wait()
        o0 = pltpu.async_copy(rows0, out0.at[pl.ds(base, per_w)], so)
        h1.wait()
        o1 = pltpu.async_copy(rows1, out1.at[pl.ds(base, per_w)], so)
        o0.wait()
        o1.wait()

    return gather


def kernel(kwids, kw_dist_adj, mask, word_embed_table):
    vocab, dim = word_embed_table.shape
    half = dim // 2
    idx = kwids.reshape(-1)
    gather = _build_gather(idx.shape[0], dim, vocab)
    o0, o1 = gather(word_embed_table[:, :half], word_embed_table[:, half:],
                    idx)
    kw_embed = jnp.concatenate([o0, o1], axis=-1).reshape(
        kwids.shape + (dim,))
    return (kw_embed, kw_dist_adj, mask)


# speedup vs baseline: 7.6137x; 2.0920x over previous
import functools

import jax
import jax.numpy as jnp
from jax import lax
from jax.experimental import pallas as pl
from jax.experimental.pallas import tpu as pltpu
from jax.experimental.pallas import tpu_sc as plsc

LANES = 16
CHUNK = 400


def _splat(x, dtype=jnp.int32):
    return jnp.full((LANES,), x, dtype)


@functools.lru_cache(maxsize=None)
def _build_gather(n_idx: int, dim: int, vocab: int):
    info = plsc.get_sparse_core_info()
    nw = info.num_cores * info.num_subcores
    assert n_idx % nw == 0
    per_w = n_idx // nw
    n_chunks = per_w // CHUNK
    mesh = plsc.VectorSubcoreMesh(core_axis_name="c", subcore_axis_name="s")

    @functools.partial(
        pl.kernel,
        mesh=mesh,
        out_type=jax.ShapeDtypeStruct((n_idx, dim), jnp.float32),
        compiler_params=pltpu.CompilerParams(needs_layout_passes=False),
        scratch_types=[
            pltpu.VMEM((per_w,), jnp.int32),
            pltpu.VMEM((CHUNK,), jnp.int32),
            pltpu.VMEM((CHUNK, 2 * dim), jnp.float32),
            pltpu.VMEM((CHUNK, dim), jnp.float32),
            pltpu.SemaphoreType.DMA,
            pltpu.SemaphoreType.DMA,
        ],
    )
    def gather(t2, idx_hbm, out_hbm, idx_v, pi_v, pairs_v, out_v, sg, so):
        wid = lax.axis_index("s") * info.num_cores + lax.axis_index("c")
        base = wid * per_w
        iota = lax.iota(jnp.int32, LANES)
        pltpu.sync_copy(idx_hbm.at[pl.ds(base, per_w)], idx_v)

        for c in range(n_chunks):
            def pids(g, carry):
                sl = pl.ds(g * LANES, LANES)
                pi_v[sl] = lax.shift_right_logical(
                    idx_v[pl.ds(c * CHUNK + g * LANES, LANES)], 1)
                return carry

            lax.fori_loop(0, CHUNK // LANES, pids, 0)
            pltpu.async_copy(t2.at[pi_v], pairs_v, sg).wait()

            def row(k, carry):
                kk = _splat(c * CHUNK) + _splat(k)
                pbase = (plsc.load_gather(idx_v, [kk]) & 1) * dim
                ks = _splat(k)
                for j in range(dim // LANES):
                    cj = iota + j * LANES
                    vals = plsc.load_gather(pairs_v, [ks, pbase + cj])
                    plsc.store_scatter(out_v, [ks, cj], vals)
                return carry

            lax.fori_loop(0, CHUNK, row, 0)
            pltpu.async_copy(
                out_v, out_hbm.at[pl.ds(base + c * CHUNK, CHUNK)], so).wait()

    return gather


def kernel(kwids, kw_dist_adj, mask, word_embed_table):
    vocab, dim = word_embed_table.shape
    idx = kwids.reshape(-1)
    gather = _build_gather(idx.shape[0], dim, vocab)
    rows = gather(word_embed_table.reshape(vocab // 2, 2 * dim), idx)
    kw_embed = rows.reshape(kwids.shape + (dim,))
    return (kw_embed, kw_dist_adj, mask)


# speedup vs baseline: 8.2580x; 1.0846x over previous
import functools

import jax
import jax.numpy as jnp
from jax import lax
from jax.experimental import pallas as pl
from jax.experimental.pallas import tpu as pltpu
from jax.experimental.pallas import tpu_sc as plsc


@functools.lru_cache(maxsize=None)
def _build_gather(n_idx: int, dim: int, vocab: int):
    info = plsc.get_sparse_core_info()
    nw = info.num_cores * info.num_subcores
    assert n_idx % nw == 0
    per_w = n_idx // nw
    mesh = plsc.VectorSubcoreMesh(core_axis_name="c", subcore_axis_name="s")

    @functools.partial(
        pl.kernel,
        mesh=mesh,
        out_type=jax.ShapeDtypeStruct((n_idx, dim), jnp.float32),
        compiler_params=pltpu.CompilerParams(use_tc_tiling_on_sc=False),
        scratch_types=[
            pltpu.VMEM((per_w,), jnp.int32),
            pltpu.VMEM((per_w, dim), jnp.float32),
            pltpu.SemaphoreType.DMA,
        ],
    )
    def gather(table_hbm, idx_hbm, out_hbm, idx_v, rows_v, sem):
        wid = lax.axis_index("s") * info.num_cores + lax.axis_index("c")
        base = wid * per_w
        pltpu.sync_copy(idx_hbm.at[pl.ds(base, per_w)], idx_v)
        pltpu.async_copy(table_hbm.at[idx_v], rows_v, sem).wait()
        pltpu.sync_copy(rows_v, out_hbm.at[pl.ds(base, per_w)])

    return gather


def kernel(kwids, kw_dist_adj, mask, word_embed_table):
    vocab, dim = word_embed_table.shape
    idx = kwids.reshape(-1)
    gather = _build_gather(idx.shape[0], dim, vocab)
    rows = gather(word_embed_table, idx)
    kw_embed = rows.reshape(kwids.shape + (dim,))
    return (kw_embed, kw_dist_adj, mask)


# speedup vs baseline: 8.9248x; 1.0807x over previous
import functools

import jax
import jax.numpy as jnp
from jax import lax
from jax.experimental import pallas as pl
from jax.experimental.pallas import tpu as pltpu
from jax.experimental.pallas import tpu_sc as plsc

CHUNK = 800


@functools.lru_cache(maxsize=None)
def _build_gather(n_idx: int, pdim: int, vocab: int):
    info = plsc.get_sparse_core_info()
    nw = info.num_cores * info.num_subcores
    assert n_idx % nw == 0
    per_w = n_idx // nw
    n_chunks = per_w // CHUNK
    mesh = plsc.VectorSubcoreMesh(core_axis_name="c", subcore_axis_name="s")

    @functools.partial(
        pl.kernel,
        mesh=mesh,
        out_type=jax.ShapeDtypeStruct((n_idx, pdim), jnp.float32),
        scratch_types=[
            pltpu.VMEM((per_w,), jnp.int32),
            pltpu.VMEM((CHUNK, pdim), jnp.float32),
            pltpu.SemaphoreType.DMA,
        ],
    )
    def gather(table_hbm, idx_hbm, out_hbm, idx_v, rows_v, sem):
        wid = lax.axis_index("s") * info.num_cores + lax.axis_index("c")
        base = wid * per_w
        pltpu.sync_copy(idx_hbm.at[pl.ds(base, per_w)], idx_v)
        for c in range(n_chunks):
            pltpu.async_copy(
                table_hbm.at[idx_v.at[pl.ds(c * CHUNK, CHUNK)]],
                rows_v, sem).wait()
            pltpu.sync_copy(
                rows_v, out_hbm.at[pl.ds(base + c * CHUNK, CHUNK)])

    return gather


def kernel(kwids, kw_dist_adj, mask, word_embed_table):
    vocab, dim = word_embed_table.shape
    padded = jnp.pad(word_embed_table, ((0, 0), (0, 128 - dim)))
    idx = kwids.reshape(-1)
    gather = _build_gather(idx.shape[0], 128, vocab)
    rows = gather(padded, idx)
    kw_embed = rows[:, :dim].reshape(kwids.shape + (dim,))
    return (kw_embed, kw_dist_adj, mask)
